# baseline (speedup 1.0000x reference)
import jax
import jax.numpy as jnp
from jax.experimental import pallas as pl
from jax.experimental.pallas import tpu as pltpu


def _add_body(t_ref, k_ref, o_ref):
    kp_col = k_ref[...].T
    o_ref[...] = t_ref[...] + kp_col[None]


def kernel(src, mask, pos_embed, tgt, tgt_mask, class_feature, kp_template):
    S, B, D = tgt.shape
    t_t = jnp.swapaxes(tgt, 1, 2)
    kp2 = kp_template.reshape(1, D)

    BLK_S = 15
    BLK_B = B // 2
    grid = ((S + BLK_S - 1) // BLK_S, 2)
    out_t = pl.pallas_call(
        _add_body,
        grid=grid,
        compiler_params=pltpu.CompilerParams(dimension_semantics=("parallel", "parallel"), vmem_limit_bytes=63 * 1024 * 1024),
        in_specs=[
            pl.BlockSpec((BLK_S, D, BLK_B), lambda i, j: (i, 0, j)),
            pl.BlockSpec((1, D), lambda i, j: (0, 0)),
        ],
        out_specs=pl.BlockSpec((BLK_S, D, BLK_B), lambda i, j: (i, 0, j)),
        out_shape=jax.ShapeDtypeStruct((S, D, B), tgt.dtype),
    )(t_t, kp2)
    return jnp.swapaxes(out_t, 1, 2)


# speedup vs baseline: 1.0623x; 1.0623x over previous
import jax
import jax.numpy as jnp
from jax.experimental import pallas as pl
from jax.experimental.pallas import tpu as pltpu


def _add_body(t_ref, k_ref, o_ref):
    kp_col = k_ref[...].T
    o_ref[...] = t_ref[...] + kp_col[None]


def kernel(src, mask, pos_embed, tgt, tgt_mask, class_feature, kp_template):
    S, B, D = tgt.shape
    t_t = jnp.swapaxes(tgt, 1, 2)
    kp2 = kp_template.reshape(1, D)

    BLK_S = 15
    grid = ((S + BLK_S - 1) // BLK_S,)
    out_t = pl.pallas_call(
        _add_body,
        grid=grid,
        compiler_params=pltpu.CompilerParams(dimension_semantics=("parallel",), vmem_limit_bytes=63 * 1024 * 1024),
        in_specs=[
            pl.BlockSpec((BLK_S, D, B), lambda i: (i, 0, 0)),
            pl.BlockSpec((1, D), lambda i: (0, 0)),
        ],
        out_specs=pl.BlockSpec((BLK_S, D, B), lambda i: (i, 0, 0)),
        out_shape=jax.ShapeDtypeStruct((S, D, B), tgt.dtype),
    )(t_t, kp2)
    return jnp.swapaxes(out_t, 1, 2)
